# baseline (speedup 1.0000x reference)
import functools

import jax
import jax.numpy as jnp
from jax.experimental import pallas as pl
from jax.experimental.pallas import tpu as pltpu
from jax.experimental.pallas import tpu_sc as plsc

D_MODEL = 4096
D_HID = 2048
NUM_EXPERTS = 64
TOP_K = 8
NUM_TOKENS = 16384

BT = 512
GI = NUM_TOKENS // BT

_SIGN_LOW = 0x7FFFFFFF
_IDX_MASK = NUM_EXPERTS - 1
_VAL_MASK = -NUM_EXPERTS
_KEY_MIN = -(2 ** 31)

SC_TILES = 32
SC_CHUNK = NUM_TOKENS // SC_TILES
SC_LANES = 16


def _to_key(f):
    s = jax.lax.bitcast_convert_type(f, jnp.int32)
    return s ^ (jax.lax.shift_right_arithmetic(s, 31) & _SIGN_LOW)


def _gating_body(x_ref, w1_ref, b1_ref, w2_ref, b2_ref, keys_ref):
    h = jnp.dot(x_ref[...], w1_ref[...],
                preferred_element_type=jnp.float32)
    h = jnp.maximum(h + b1_ref[...], 0.0)
    logits = jnp.dot(h, w2_ref[...],
                     preferred_element_type=jnp.float32) + b2_ref[...]

    iota = jax.lax.broadcasted_iota(jnp.int32, (BT, NUM_EXPERTS), 1)
    key = (_to_key(logits) & _VAL_MASK) | (_IDX_MASK - iota)
    keys_ref[...] = key.T


def _unpack_val(k):
    vk = k & _VAL_MASK
    s = vk ^ (jax.lax.shift_right_arithmetic(vk, 31) & _SIGN_LOW)
    return jax.lax.bitcast_convert_type(s, jnp.float32)


def _topk_sc(keys_t):
    vector_mesh = plsc.VectorSubcoreMesh(
        core_axis_name="core", subcore_axis_name="subcore")

    @pl.kernel(
        out_type=[
            jax.ShapeDtypeStruct((TOP_K, NUM_TOKENS), jnp.float32),
            jax.ShapeDtypeStruct((TOP_K, NUM_TOKENS), jnp.int32),
            jax.ShapeDtypeStruct((SC_TILES, NUM_EXPERTS, SC_LANES),
                                 jnp.float32),
        ],
        mesh=vector_mesh,
        scratch_types=[pltpu.VMEM((NUM_EXPERTS, SC_LANES), jnp.float32)],
    )
    def sc_kernel(keys_hbm, g_hbm, i_hbm, p_hbm, tmp_ref):
        def body(k_vmem, g_vmem, i_vmem, p_vmem):
            for e in range(NUM_EXPERTS):
                p_vmem[0, e, :] = jnp.zeros((SC_LANES,), jnp.float32)

            @pl.loop(0, SC_CHUNK // SC_LANES)
            def _(g):
                sl = pl.ds(g * SC_LANES, SC_LANES)
                m = [jnp.full((SC_LANES,), _KEY_MIN, jnp.int32)
                     for _ in range(TOP_K)]
                for e in range(NUM_EXPERTS):
                    v = k_vmem[e, sl]
                    for r in range(TOP_K):
                        hi = jnp.maximum(m[r], v)
                        v = jnp.minimum(m[r], v)
                        m[r] = hi
                vals = []
                for r in range(TOP_K):
                    i_vmem[r, sl] = _IDX_MASK - (m[r] & _IDX_MASK)
                    vals.append(_unpack_val(m[r]))
                row_max = vals[0]
                es = [jnp.exp(v - row_max) for v in vals]
                tot = es[0]
                for r in range(1, TOP_K):
                    tot = tot + es[r]
                for r in range(TOP_K):
                    g_vmem[r, sl] = es[r] / tot

                full_tot = jnp.zeros((SC_LANES,), jnp.float32)
                for e in range(NUM_EXPERTS):
                    ee = jnp.exp(_unpack_val(k_vmem[e, sl]) - row_max)
                    tmp_ref[e, :] = ee
                    full_tot = full_tot + ee
                inv = 1.0 / full_tot
                for e in range(NUM_EXPERTS):
                    p_vmem[0, e, :] = p_vmem[0, e, :] + tmp_ref[e, :] * inv

        pltpu.emit_pipeline(
            body,
            grid=(SC_TILES,),
            in_specs=[pl.BlockSpec((NUM_EXPERTS, SC_CHUNK),
                                   index_map=lambda i: (0, i))],
            out_specs=[pl.BlockSpec((TOP_K, SC_CHUNK),
                                    index_map=lambda i: (0, i)),
                       pl.BlockSpec((TOP_K, SC_CHUNK),
                                    index_map=lambda i: (0, i)),
                       pl.BlockSpec((1, NUM_EXPERTS, SC_LANES),
                                    index_map=lambda i: (i, 0, 0))],
            core_axis_name=("core", "subcore"),
            dimension_semantics=(pltpu.PARALLEL,),
        )(keys_hbm, g_hbm, i_hbm, p_hbm)

    return sc_kernel(keys_t)


def _loss_body(p_ref, loss_ref):
    per_expert = jnp.sum(p_ref[...], axis=0)
    expert_sums = jnp.sum(per_expert, axis=1, keepdims=True)
    expert_probs = expert_sums * (1.0 / NUM_TOKENS)
    log_input = jnp.log(expert_probs + 1e-08)
    target = 1.0 / NUM_EXPERTS
    loss_ref[...] = jnp.sum(target * (jnp.log(target) - log_input),
                            keepdims=True)


@functools.partial(jax.jit, static_argnames=("interpret",))
def _gating(x, w1, b1, w2, b2, interpret=False):
    keys_t = pl.pallas_call(
        _gating_body,
        grid=(GI,),
        in_specs=[
            pl.BlockSpec((BT, D_MODEL), lambda i: (i, 0)),
            pl.BlockSpec((D_MODEL, D_HID), lambda i: (0, 0)),
            pl.BlockSpec((1, D_HID), lambda i: (0, 0)),
            pl.BlockSpec((D_HID, NUM_EXPERTS), lambda i: (0, 0)),
            pl.BlockSpec((1, NUM_EXPERTS), lambda i: (0, 0)),
        ],
        out_specs=pl.BlockSpec((NUM_EXPERTS, BT), lambda i: (0, i)),
        out_shape=jax.ShapeDtypeStruct((NUM_EXPERTS, NUM_TOKENS), jnp.int32),
        interpret=interpret,
    )(x, w1, b1, w2, b2)
    g_t, i_t, psums = _topk_sc(keys_t)
    loss = pl.pallas_call(
        _loss_body,
        out_shape=jax.ShapeDtypeStruct((1, 1), jnp.float32),
        interpret=interpret,
    )(psums)
    return g_t.T, i_t.T, loss


def kernel(x, training, W1, b1, W2, b2, interpret=False):
    del training
    gates, idx, loss = _gating(x, W1, b1.reshape(1, D_HID),
                               W2, b2.reshape(1, NUM_EXPERTS),
                               interpret=interpret)
    return gates, idx, loss.reshape(())
